# baseline (speedup 1.0000x reference)
import functools

import jax
import jax.numpy as jnp
from jax import lax
from jax.experimental import pallas as pl
from jax.experimental.pallas import tpu as pltpu
from jax.experimental.pallas import tpu_sc as plsc

N_CAND = 1024
SEQ = 128
D = 256
NUM_FEAT = 32

_INFO = plsc.get_sparse_core_info()
_NC = _INFO.num_cores
_NS = _INFO.num_subcores
_NW = _NC * _NS
_BPW = N_CAND // _NW
_L = 16

_FEATS = (
    (1, 0, 0 * D),
    (2, 0, 1 * D),
    (4, 0, 2 * D),
    (5, 0, 3 * D),
    (0, 1, 4 * D),
    (3, 1, 5 * D),
)
_OUT_W = 6 * D + NUM_FEAT


def _body(bl_hbm, at_hbm, ct_hbm, nf_hbm, out_hbm,
          c_v, idx_v, bufs, nf_v, gsem, wsem):
  wid = lax.axis_index("s") * _NC + lax.axis_index("c")
  base = wid * _BPW

  lanes = lax.iota(jnp.int32, _L)
  copies = []
  for f, (col, tab, _off) in enumerate(_FEATS):
    mul = SEQ if tab == 0 else SEQ + 2
    bias = 0 if tab == 0 else 2
    pltpu.sync_copy(ct_hbm.at[col, pl.ds(base, _BPW)], c_v[f])
    for h in range(_BPW // _L):
      rows16 = lanes + (h * _L)
      cvals = c_v[f][pl.ds(h * _L, _L)]
      flat = (rows16 + base) * mul + cvals + bias
      idx_v[f][pl.ds(h * _L, _L)] = flat
    table = bl_hbm if tab == 0 else at_hbm
    copies.append(pltpu.async_copy(table.at[idx_v[f]], bufs[f], gsem))

  pltpu.sync_copy(nf_hbm.at[pl.ds(base, _BPW)], nf_v)
  pltpu.sync_copy(nf_v, out_hbm.at[pl.ds(base, _BPW), pl.ds(6 * D, NUM_FEAT)])

  wcopies = []
  for f, (_col, _tab, off) in enumerate(_FEATS):
    copies[f].wait()
    wcopies.append(pltpu.async_copy(
        bufs[f], out_hbm.at[pl.ds(base, _BPW), pl.ds(off, D)], wsem))
  for wc in wcopies:
    wc.wait()


@jax.jit
def _run(bl_flat, at_flat, ct, nf):
  mesh = plsc.VectorSubcoreMesh(core_axis_name="c", subcore_axis_name="s")
  scratch = [
      [pltpu.VMEM((_BPW,), jnp.int32) for _ in _FEATS],
      [pltpu.VMEM((_BPW,), jnp.int32) for _ in _FEATS],
      [pltpu.VMEM((_BPW, D), jnp.float32) for _ in _FEATS],
      pltpu.VMEM((_BPW, NUM_FEAT), jnp.float32),
      pltpu.SemaphoreType.DMA,
      pltpu.SemaphoreType.DMA,
  ]
  fn = pl.kernel(
      _body,
      out_type=jax.ShapeDtypeStruct((N_CAND, _OUT_W), jnp.float32),
      mesh=mesh,
      scratch_types=scratch,
  )
  return fn(bl_flat, at_flat, ct, nf)


def kernel(candidates, candidate_numeric_features, stacked_bi_lstm_output,
           stacked_attended_nodes):
  ct = candidates[0].T.copy()
  nf = candidate_numeric_features[0]
  bl_flat = stacked_bi_lstm_output.reshape(N_CAND * SEQ, D)
  at_flat = stacked_attended_nodes.reshape(N_CAND * (SEQ + 2), D)
  out = _run(bl_flat, at_flat, ct, nf)
  return out[None]


# speedup vs baseline: 1.0037x; 1.0037x over previous
import functools

import jax
import jax.numpy as jnp
from jax import lax
from jax.experimental import pallas as pl
from jax.experimental.pallas import tpu as pltpu
from jax.experimental.pallas import tpu_sc as plsc

N_CAND = 1024
SEQ = 128
D = 256
NUM_FEAT = 32

_INFO = plsc.get_sparse_core_info()
_NC = _INFO.num_cores
_NS = _INFO.num_subcores
_NW = _NC * _NS
_BPW = N_CAND // _NW
_L = 16

_FEATS = (
    (1, 0, 0 * D),
    (2, 0, 1 * D),
    (4, 0, 2 * D),
    (5, 0, 3 * D),
    (0, 1, 4 * D),
    (3, 1, 5 * D),
)
_OUT_W = 6 * D + NUM_FEAT


def _body(bl_hbm, at_hbm, ct_hbm, nf_hbm, out_hbm,
          c_v, idx_v, bufs, nf_v, gsem, wsem):
  wid = lax.axis_index("s") * _NC + lax.axis_index("c")
  base = wid * _BPW

  lanes = lax.iota(jnp.int32, _L)
  copies = []
  for f, (col, tab, _off) in enumerate(_FEATS):
    mul = SEQ if tab == 0 else SEQ + 2
    bias = 0 if tab == 0 else 2
    pltpu.sync_copy(ct_hbm.at[col, pl.ds(base, _BPW)], c_v[f])
    for h in range(_BPW // _L):
      rows16 = lanes + (h * _L)
      cvals = c_v[f][pl.ds(h * _L, _L)]
      flat = (rows16 + base) * mul + cvals + bias
      idx_v[f][pl.ds(h * _L, _L)] = flat
    table = bl_hbm if tab == 0 else at_hbm
    copies.append(pltpu.async_copy(table.at[idx_v[f]], bufs[f], gsem))

  pltpu.sync_copy(nf_hbm.at[pl.ds(base, _BPW)], nf_v)
  pltpu.sync_copy(nf_v, out_hbm.at[pl.ds(base, _BPW), pl.ds(6 * D, NUM_FEAT)])

  wcopies = []
  for f, (_col, _tab, off) in enumerate(_FEATS):
    copies[f].wait()
    wcopies.append(pltpu.async_copy(
        bufs[f], out_hbm.at[pl.ds(base, _BPW), pl.ds(off, D)], wsem))
  for wc in wcopies:
    wc.wait()


@jax.jit
def _run(bl_flat, at_flat, ct, nf):
  mesh = plsc.VectorSubcoreMesh(core_axis_name="c", subcore_axis_name="s")
  scratch = [
      [pltpu.VMEM((_BPW,), jnp.int32) for _ in _FEATS],
      [pltpu.VMEM((_BPW,), jnp.int32) for _ in _FEATS],
      [pltpu.VMEM((_BPW, D), jnp.float32) for _ in _FEATS],
      pltpu.VMEM((_BPW, NUM_FEAT), jnp.float32),
      pltpu.SemaphoreType.DMA,
      pltpu.SemaphoreType.DMA,
  ]
  fn = pl.kernel(
      _body,
      out_type=jax.ShapeDtypeStruct((N_CAND, _OUT_W), jnp.float32),
      mesh=mesh,
      scratch_types=scratch,
      compiler_params=pltpu.CompilerParams(use_tc_tiling_on_sc=True),
  )
  return fn(bl_flat, at_flat, ct, nf)


def kernel(candidates, candidate_numeric_features, stacked_bi_lstm_output,
           stacked_attended_nodes):
  ct = candidates[0].T.copy()
  nf = candidate_numeric_features[0]
  bl_flat = stacked_bi_lstm_output.reshape(N_CAND * SEQ, D)
  at_flat = stacked_attended_nodes.reshape(N_CAND * (SEQ + 2), D)
  out = _run(bl_flat, at_flat, ct, nf)
  return out[None]


# speedup vs baseline: 7.4213x; 7.3937x over previous
import functools

import jax
import jax.numpy as jnp
from jax import lax
from jax.experimental import pallas as pl
from jax.experimental.pallas import tpu as pltpu
from jax.experimental.pallas import tpu_sc as plsc

N_CAND = 1024
SEQ = 128
D = 256
NUM_FEAT = 32

_INFO = plsc.get_sparse_core_info()
_NC = _INFO.num_cores
_NS = _INFO.num_subcores
_NW = _NC * _NS
_BPW = N_CAND // _NW
_L = 16

_FEATS = (
    (1, 0, 0 * D),
    (2, 0, 1 * D),
    (4, 0, 2 * D),
    (5, 0, 3 * D),
    (0, 1, 4 * D),
    (3, 1, 5 * D),
)
_OUT_W = 6 * D + NUM_FEAT


def _body(bl_hbm, at_hbm, ct_hbm, nf_hbm, out_hbm,
          c_v, idx_v, bufs, nf_v, gsem, wsem):
  wid = lax.axis_index("s") * _NC + lax.axis_index("c")
  base = wid * _BPW

  lanes = lax.iota(jnp.int32, _L)
  copies = []
  for f, (col, tab, _off) in enumerate(_FEATS):
    pltpu.sync_copy(ct_hbm.at[col, pl.ds(base, _BPW)], c_v[f])
    for h in range(_BPW // _L):
      rows16 = lanes + (h * _L)
      cvals = c_v[f][pl.ds(h * _L, _L)]
      if tab == 0:
        flat = (base + rows16) * SEQ + cvals
      else:
        flat = (cvals + 2) * N_CAND + base + rows16
      idx_v[f][pl.ds(h * _L, _L)] = flat
    table = bl_hbm if tab == 0 else at_hbm
    copies.append(pltpu.async_copy(table.at[idx_v[f]], bufs[f], gsem))

  pltpu.sync_copy(nf_hbm.at[pl.ds(base, _BPW)], nf_v)
  pltpu.sync_copy(nf_v, out_hbm.at[pl.ds(base, _BPW), pl.ds(6 * D, NUM_FEAT)])

  wcopies = []
  for f, (_col, _tab, off) in enumerate(_FEATS):
    copies[f].wait()
    wcopies.append(pltpu.async_copy(
        bufs[f], out_hbm.at[pl.ds(base, _BPW), pl.ds(off, D)], wsem))
  for wc in wcopies:
    wc.wait()


@jax.jit
def _run(bl_flat, at_flat, ct, nf):
  mesh = plsc.VectorSubcoreMesh(core_axis_name="c", subcore_axis_name="s")
  scratch = [
      [pltpu.VMEM((_BPW,), jnp.int32) for _ in _FEATS],
      [pltpu.VMEM((_BPW,), jnp.int32) for _ in _FEATS],
      [pltpu.VMEM((_BPW, D), jnp.float32) for _ in _FEATS],
      pltpu.VMEM((_BPW, NUM_FEAT), jnp.float32),
      pltpu.SemaphoreType.DMA,
      pltpu.SemaphoreType.DMA,
  ]
  fn = pl.kernel(
      _body,
      out_type=jax.ShapeDtypeStruct((N_CAND, _OUT_W), jnp.float32),
      mesh=mesh,
      scratch_types=scratch,
      compiler_params=pltpu.CompilerParams(use_tc_tiling_on_sc=True),
  )
  return fn(bl_flat, at_flat, ct, nf)


def kernel(candidates, candidate_numeric_features, stacked_bi_lstm_output,
           stacked_attended_nodes):
  ct = candidates[0].T
  nf = candidate_numeric_features[0]
  bl_flat = stacked_bi_lstm_output.reshape(N_CAND * SEQ, D)
  at_flat = stacked_attended_nodes[0].transpose(1, 0, 2).reshape(
      (SEQ + 2) * N_CAND, D)
  out = _run(bl_flat, at_flat, ct, nf)
  return out[None]


# speedup vs baseline: 7.6176x; 1.0265x over previous
import jax
import jax.numpy as jnp
from jax import lax
from jax.experimental import pallas as pl
from jax.experimental.pallas import tpu as pltpu
from jax.experimental.pallas import tpu_sc as plsc

N_CAND = 1024
SEQ = 128
D = 256
NUM_FEAT = 32

_INFO = plsc.get_sparse_core_info()
_NC = _INFO.num_cores
_NS = _INFO.num_subcores
_NW = _NC * _NS
_BPW = N_CAND // _NW
_L = 16

_BL_COLS = (1, 2, 4, 5)
_AT_COLS = (0, 3)
_GATHER_W = (len(_BL_COLS) + len(_AT_COLS)) * D


def _body(bl_hbm, at_hbm, ct_hbm, nf_hbm, out_hbm,
          c_v, bl_idx, at_idx, bl_buf, at_buf, nf_v, csem, gsem, wsem):
  wid = lax.axis_index("s") * _NC + lax.axis_index("c")
  base = wid * _BPW

  ccopies = []
  for f, col in enumerate(_BL_COLS + _AT_COLS):
    ccopies.append(pltpu.async_copy(
        ct_hbm.at[pl.ds(col * N_CAND + base, _BPW)], c_v[f], csem))
  for cc in ccopies:
    cc.wait()

  lanes = lax.iota(jnp.int32, _L)
  for f, _col in enumerate(_BL_COLS):
    for h in range(_BPW // _L):
      rows16 = lanes + (h * _L)
      cvals = c_v[f][pl.ds(h * _L, _L)]
      bl_idx[pl.ds(f * _BPW + h * _L, _L)] = (base + rows16) * SEQ + cvals
  for f, _col in enumerate(_AT_COLS):
    for h in range(_BPW // _L):
      rows16 = lanes + (h * _L)
      cvals = c_v[len(_BL_COLS) + f][pl.ds(h * _L, _L)]
      at_idx[pl.ds(f * _BPW + h * _L, _L)] = (
          (cvals + 2) * N_CAND + base + rows16)

  blc = pltpu.async_copy(bl_hbm.at[bl_idx], bl_buf, gsem)
  atc = pltpu.async_copy(at_hbm.at[at_idx], at_buf, gsem)

  pltpu.sync_copy(nf_hbm.at[pl.ds(base, _BPW)], nf_v)
  pltpu.sync_copy(
      nf_v, out_hbm.at[pl.ds(base, _BPW), pl.ds(len(_BL_COLS + _AT_COLS) * D,
                                                NUM_FEAT)])

  wcopies = []
  blc.wait()
  for f in range(len(_BL_COLS)):
    wcopies.append(pltpu.async_copy(
        bl_buf.at[pl.ds(f * _BPW, _BPW)],
        out_hbm.at[pl.ds(base, _BPW), pl.ds(f * D, D)], wsem))
  atc.wait()
  for f in range(len(_AT_COLS)):
    wcopies.append(pltpu.async_copy(
        at_buf.at[pl.ds(f * _BPW, _BPW)],
        out_hbm.at[pl.ds(base, _BPW), pl.ds((len(_BL_COLS) + f) * D, D)],
        wsem))
  for wc in wcopies:
    wc.wait()


@jax.jit
def _run(bl_flat, at_flat, cflat, nf):
  mesh = plsc.VectorSubcoreMesh(core_axis_name="c", subcore_axis_name="s")
  nbl = len(_BL_COLS) * _BPW
  nat = len(_AT_COLS) * _BPW
  scratch = [
      [pltpu.VMEM((_BPW,), jnp.int32) for _ in range(6)],
      pltpu.VMEM((nbl,), jnp.int32),
      pltpu.VMEM((nat,), jnp.int32),
      pltpu.VMEM((nbl, D), jnp.float32),
      pltpu.VMEM((nat, D), jnp.float32),
      pltpu.VMEM((_BPW, NUM_FEAT), jnp.float32),
      pltpu.SemaphoreType.DMA,
      pltpu.SemaphoreType.DMA,
      pltpu.SemaphoreType.DMA,
  ]
  fn = pl.kernel(
      _body,
      out_type=jax.ShapeDtypeStruct((N_CAND, _GATHER_W + NUM_FEAT),
                                    jnp.float32),
      mesh=mesh,
      scratch_types=scratch,
  )
  return fn(bl_flat, at_flat, cflat, nf)


def kernel(candidates, candidate_numeric_features, stacked_bi_lstm_output,
           stacked_attended_nodes):
  cflat = candidates[0].T.reshape(6 * N_CAND)
  nf = candidate_numeric_features[0]
  bl_flat = stacked_bi_lstm_output.reshape(N_CAND * SEQ, D)
  at_flat = stacked_attended_nodes[0].transpose(1, 0, 2).reshape(
      (SEQ + 2) * N_CAND, D)
  return _run(bl_flat, at_flat, cflat, nf)[None]
